# Initial kernel scaffold; baseline (speedup 1.0000x reference)
import jax
import jax.numpy as jnp
from jax.experimental import pallas as pl


def kernel(x, grads, background, seg):
    raise NotImplementedError("write your pallas kernel here")


# speedup vs baseline: 181.7720x; 181.7720x over previous
import jax
import jax.numpy as jnp
from jax import lax
from jax.experimental import pallas as pl
from jax.experimental.pallas import tpu as pltpu
from jax.experimental.pallas import tpu_sc as plsc

B = 128
N = 32768
S = 1024
K = 50
NC, NS = 2, 16
NW = NC * NS
RPW = B // NW
L = 16
VPR = N // L


def _segsum_body(grads_hbm, seg_hbm, scores_hbm, gbuf, sbuf, scores_v):
    wid = lax.axis_index("s") * NC + lax.axis_index("c")

    def row_body(r, carry):
        row = wid * RPW + r
        pltpu.sync_copy(grads_hbm.at[row], gbuf)
        pltpu.sync_copy(seg_hbm.at[row], sbuf)

        def zero_body(j, c):
            scores_v[pl.ds(j * L, L)] = jnp.zeros((L,), jnp.float32)
            return c
        lax.fori_loop(0, S // L, zero_body, 0)

        def acc_body(i, c):
            sl = pl.ds(i * L, L)
            plsc.addupdate_scatter(scores_v, [sbuf[sl]], gbuf[sl])
            return c
        lax.fori_loop(0, VPR, acc_body, 0)

        pltpu.sync_copy(scores_v, scores_hbm.at[row])
        return carry
    lax.fori_loop(0, RPW, row_body, 0)


def _topk_mask_body(scores_ref, mask_ref):
    s = scores_ref[...]
    b = lax.bitcast_convert_type(s, jnp.int32)
    key = b ^ (lax.shift_right_arithmetic(b, 31) & jnp.int32(0x7FFFFFFF))
    min32 = jnp.int32(-(2 ** 31))

    def bit_body(i, tu):
        bit = lax.shift_left(jnp.int32(1), 31 - i)
        cand_u = tu | bit
        cand_s = cand_u ^ min32
        cnt = jnp.sum((key >= cand_s).astype(jnp.int32), axis=1, keepdims=True)
        return jnp.where(cnt >= K, cand_u, tu)

    tu = lax.fori_loop(0, 32, bit_body, jnp.zeros((B, 1), jnp.int32))
    ts = tu ^ min32
    gt = key > ts
    eq = key == ts
    cnt_gt = jnp.sum(gt.astype(jnp.int32), axis=1, keepdims=True)
    need = (K - cnt_gt).astype(jnp.float32)
    ii = lax.broadcasted_iota(jnp.int32, (S, S), 0)
    jj = lax.broadcasted_iota(jnp.int32, (S, S), 1)
    tri = (ii <= jj).astype(jnp.float32)
    cum = lax.dot_general(eq.astype(jnp.float32), tri,
                          (((1,), (0,)), ((), ())),
                          preferred_element_type=jnp.float32)
    mask = jnp.logical_or(gt, jnp.logical_and(eq, cum <= need))
    mask_ref[...] = mask.astype(jnp.float32)


def _blend_body(x_hbm, bg_hbm, seg_hbm, mask_hbm, out_hbm,
                xbuf, bbuf, sbuf, mask_v):
    wid = lax.axis_index("s") * NC + lax.axis_index("c")

    def row_body(r, carry):
        row = wid * RPW + r
        pltpu.sync_copy(mask_hbm.at[row], mask_v)
        pltpu.sync_copy(x_hbm.at[row], xbuf)
        pltpu.sync_copy(bg_hbm.at[row], bbuf)
        pltpu.sync_copy(seg_hbm.at[row], sbuf)

        def v_body(i, c):
            sl = pl.ds(i * L, L)
            xv = xbuf[sl]
            bv = bbuf[sl]
            m = plsc.load_gather(mask_v, [sbuf[sl]])
            d = (xv - bv) * m
            bbuf[sl] = bv + d
            xbuf[sl] = xv - d
            return c
        lax.fori_loop(0, VPR, v_body, 0)

        pltpu.sync_copy(bbuf, out_hbm.at[row])
        pltpu.sync_copy(xbuf, out_hbm.at[B + row])
        return carry
    lax.fori_loop(0, RPW, row_body, 0)


def kernel(x, grads, background, seg):
    mesh = plsc.VectorSubcoreMesh(core_axis_name="c", subcore_axis_name="s",
                                  num_cores=NC, num_subcores=NS)
    sc_params = pltpu.CompilerParams(use_tc_tiling_on_sc=False,
                                     needs_layout_passes=False)
    segsum = pl.kernel(
        _segsum_body,
        out_type=jax.ShapeDtypeStruct((B, S), jnp.float32),
        mesh=mesh,
        compiler_params=sc_params,
        scratch_types=[
            pltpu.VMEM((N,), jnp.float32),
            pltpu.VMEM((N,), jnp.int32),
            pltpu.VMEM((S,), jnp.float32),
        ],
    )
    scores = segsum(grads, seg)

    seg_mask = pl.pallas_call(
        _topk_mask_body,
        out_shape=jax.ShapeDtypeStruct((B, S), jnp.float32),
    )(scores)

    blend = pl.kernel(
        _blend_body,
        out_type=jax.ShapeDtypeStruct((2 * B, N), jnp.float32),
        mesh=mesh,
        compiler_params=sc_params,
        scratch_types=[
            pltpu.VMEM((N,), jnp.float32),
            pltpu.VMEM((N,), jnp.float32),
            pltpu.VMEM((N,), jnp.int32),
            pltpu.VMEM((S,), jnp.float32),
        ],
    )
    out = blend(x, background, seg, seg_mask)
    return out.reshape(2, B, N)


# speedup vs baseline: 220.1092x; 1.2109x over previous
import jax
import jax.numpy as jnp
from jax import lax
from jax.experimental import pallas as pl
from jax.experimental.pallas import tpu as pltpu
from jax.experimental.pallas import tpu_sc as plsc

B = 128
N = 32768
S = 1024
K = 50
NC, NS = 2, 16
NW = NC * NS
RPW = B // NW
L = 16
CPR = 4
C = N // CPR
CPW = RPW * CPR


def _segsum_body(grads_hbm, seg_hbm, scores_hbm, gbuf, sbuf, scores_v,
                 sem0, sem1):
    wid = lax.axis_index("s") * NC + lax.axis_index("c")
    row0 = wid * RPW
    sems = (sem0, sem1)

    def zero():
        def zb(j, c):
            scores_v[pl.ds(j * L, L)] = jnp.zeros((L,), jnp.float32)
            return c
        lax.fori_loop(0, S // L, zb, 0)

    def issue(k):
        r, j = divmod(k, CPR)
        slot = k % 2
        sl = pl.ds(j * C, C)
        return (
            pltpu.async_copy(grads_hbm.at[row0 + r, sl], gbuf.at[slot],
                             sems[slot]),
            pltpu.async_copy(seg_hbm.at[row0 + r, sl], sbuf.at[slot],
                             sems[slot]),
        )

    zero()
    descs = {0: issue(0)}
    for k in range(CPW):
        if k + 1 < CPW:
            descs[k + 1] = issue(k + 1)
        for d in descs.pop(k):
            d.wait()
        slot = k % 2
        gb, sb = gbuf.at[slot], sbuf.at[slot]

        def ab(i, c, gb=gb, sb=sb):
            sl = pl.ds(i * L, L)
            plsc.addupdate_scatter(scores_v, [sb[sl]], gb[sl])
            return c
        lax.fori_loop(0, C // L, ab, 0)

        r, j = divmod(k, CPR)
        if j == CPR - 1:
            pltpu.sync_copy(scores_v, scores_hbm.at[row0 + r])
            if k + 1 < CPW:
                zero()


def _topk_mask_body(scores_ref, mask_ref):
    s = scores_ref[...]
    b = lax.bitcast_convert_type(s, jnp.int32)
    key = b ^ (lax.shift_right_arithmetic(b, 31) & jnp.int32(0x7FFFFFFF))
    min32 = jnp.int32(-(2 ** 31))

    def bit_body(i, tu):
        bit = lax.shift_left(jnp.int32(1), 31 - i)
        cand_u = tu | bit
        cand_s = cand_u ^ min32
        cnt = jnp.sum((key >= cand_s).astype(jnp.int32), axis=1,
                      keepdims=True)
        return jnp.where(cnt >= K, cand_u, tu)

    tu = lax.fori_loop(0, 32, bit_body, jnp.zeros((B, 1), jnp.int32))
    ts = tu ^ min32
    gt = key > ts
    eq = key == ts
    cnt_gt = jnp.sum(gt.astype(jnp.int32), axis=1, keepdims=True)
    need = (K - cnt_gt).astype(jnp.float32)
    ii = lax.broadcasted_iota(jnp.int32, (S, S), 0)
    jj = lax.broadcasted_iota(jnp.int32, (S, S), 1)
    tri = (ii <= jj).astype(jnp.float32)
    cum = lax.dot_general(eq.astype(jnp.float32), tri,
                          (((1,), (0,)), ((), ())),
                          preferred_element_type=jnp.float32)
    mask = jnp.logical_or(gt, jnp.logical_and(eq, cum <= need))
    mask_ref[...] = mask.astype(jnp.float32)


def _blend_body(x_hbm, bg_hbm, seg_hbm, mask_hbm, out_hbm,
                xbuf, bbuf, sbuf, rbuf, abuf, mask_v,
                semi0, semi1, semo0, semo1):
    wid = lax.axis_index("s") * NC + lax.axis_index("c")
    row0 = wid * RPW
    in_sems = (semi0, semi1)
    out_sems = (semo0, semo1)

    pltpu.sync_copy(mask_hbm.at[pl.ds(row0, RPW)], mask_v)

    def issue_in(k):
        r, j = divmod(k, CPR)
        slot = k % 2
        sl = pl.ds(j * C, C)
        row = row0 + r
        return (
            pltpu.async_copy(x_hbm.at[row, sl], xbuf.at[slot],
                             in_sems[slot]),
            pltpu.async_copy(bg_hbm.at[row, sl], bbuf.at[slot],
                             in_sems[slot]),
            pltpu.async_copy(seg_hbm.at[row, sl], sbuf.at[slot],
                             in_sems[slot]),
        )

    def issue_out(k):
        r, j = divmod(k, CPR)
        slot = k % 2
        sl = pl.ds(j * C, C)
        row = row0 + r
        return (
            pltpu.async_copy(rbuf.at[slot], out_hbm.at[row, sl],
                             out_sems[slot]),
            pltpu.async_copy(abuf.at[slot], out_hbm.at[B + row, sl],
                             out_sems[slot]),
        )

    in_descs = {0: issue_in(0)}
    out_descs = {}
    for k in range(CPW):
        if k + 1 < CPW:
            in_descs[k + 1] = issue_in(k + 1)
        for d in in_descs.pop(k):
            d.wait()
        if k - 2 in out_descs:
            for d in out_descs.pop(k - 2):
                d.wait()
        slot = k % 2
        r = k // CPR
        rv = jnp.full((L,), r, jnp.int32)
        xb, bb, sb = xbuf.at[slot], bbuf.at[slot], sbuf.at[slot]
        rb, ab_ = rbuf.at[slot], abuf.at[slot]

        def vb(i, c, xb=xb, bb=bb, sb=sb, rb=rb, ab_=ab_, rv=rv):
            sl = pl.ds(i * L, L)
            xv = xb[sl]
            bv = bb[sl]
            m = plsc.load_gather(mask_v, [rv, sb[sl]])
            d = (xv - bv) * m
            rb[sl] = bv + d
            ab_[sl] = xv - d
            return c
        lax.fori_loop(0, C // L, vb, 0)
        out_descs[k] = issue_out(k)

    for k in sorted(out_descs):
        for d in out_descs.pop(k):
            d.wait()


def kernel(x, grads, background, seg):
    mesh = plsc.VectorSubcoreMesh(core_axis_name="c", subcore_axis_name="s",
                                  num_cores=NC, num_subcores=NS)
    sc_params = pltpu.CompilerParams(use_tc_tiling_on_sc=False,
                                     needs_layout_passes=False)
    segsum = pl.kernel(
        _segsum_body,
        out_type=jax.ShapeDtypeStruct((B, S), jnp.float32),
        mesh=mesh,
        compiler_params=sc_params,
        scratch_types=[
            pltpu.VMEM((2, C), jnp.float32),
            pltpu.VMEM((2, C), jnp.int32),
            pltpu.VMEM((S,), jnp.float32),
            pltpu.SemaphoreType.DMA,
            pltpu.SemaphoreType.DMA,
        ],
    )
    scores = segsum(grads, seg)

    seg_mask = pl.pallas_call(
        _topk_mask_body,
        out_shape=jax.ShapeDtypeStruct((B, S), jnp.float32),
    )(scores)

    blend = pl.kernel(
        _blend_body,
        out_type=jax.ShapeDtypeStruct((2 * B, N), jnp.float32),
        mesh=mesh,
        compiler_params=sc_params,
        scratch_types=[
            pltpu.VMEM((2, C), jnp.float32),
            pltpu.VMEM((2, C), jnp.float32),
            pltpu.VMEM((2, C), jnp.int32),
            pltpu.VMEM((2, C), jnp.float32),
            pltpu.VMEM((2, C), jnp.float32),
            pltpu.VMEM((RPW, S), jnp.float32),
            pltpu.SemaphoreType.DMA,
            pltpu.SemaphoreType.DMA,
            pltpu.SemaphoreType.DMA,
            pltpu.SemaphoreType.DMA,
        ],
    )
    out = blend(x, background, seg, seg_mask)
    return out.reshape(2, B, N)


# speedup vs baseline: 285.6753x; 1.2979x over previous
import jax
import jax.numpy as jnp
from jax import lax
from jax.experimental import pallas as pl
from jax.experimental.pallas import tpu as pltpu
from jax.experimental.pallas import tpu_sc as plsc

B = 128
N = 32768
S = 1024
K = 50
NC, NS = 2, 16
NW = NC * NS
RPW = B // NW
L = 16
CPR = 4
C = N // CPR
CPW = RPW * CPR


def _segsum_body(grads_hbm, seg_hbm, scores_hbm, gbuf, sbuf, scores_v,
                 sem0, sem1):
    wid = lax.axis_index("s") * NC + lax.axis_index("c")
    row0 = wid * RPW
    sems = (sem0, sem1)

    def zero():
        def zb(j, c):
            scores_v[pl.ds(j * L, L)] = jnp.zeros((L,), jnp.float32)
            return c
        lax.fori_loop(0, S // L, zb, 0)

    def issue(k):
        r, j = divmod(k, CPR)
        slot = k % 2
        sl = pl.ds(j * C, C)
        return (
            pltpu.async_copy(grads_hbm.at[row0 + r, sl], gbuf.at[slot],
                             sems[slot]),
            pltpu.async_copy(seg_hbm.at[row0 + r, sl], sbuf.at[slot],
                             sems[slot]),
        )

    zero()
    descs = {0: issue(0)}
    for k in range(CPW):
        if k + 1 < CPW:
            descs[k + 1] = issue(k + 1)
        for d in descs.pop(k):
            d.wait()
        slot = k % 2
        gb, sb = gbuf.at[slot], sbuf.at[slot]

        @plsc.parallel_loop(0, C, L, unroll=8)
        def _acc(i, gb=gb, sb=sb):
            sl = pl.ds(i, L)
            plsc.addupdate_scatter(scores_v, [sb[sl]], gb[sl])

        r, j = divmod(k, CPR)
        if j == CPR - 1:
            pltpu.sync_copy(scores_v, scores_hbm.at[row0 + r])
            if k + 1 < CPW:
                zero()


def _topk_mask_body(scores_ref, mask_ref):
    s = scores_ref[...]
    b = lax.bitcast_convert_type(s, jnp.int32)
    key = b ^ (lax.shift_right_arithmetic(b, 31) & jnp.int32(0x7FFFFFFF))
    min32 = jnp.int32(-(2 ** 31))

    def bit_body(i, tu):
        bit = lax.shift_left(jnp.int32(1), 31 - i)
        cand_u = tu | bit
        cand_s = cand_u ^ min32
        cnt = jnp.sum((key >= cand_s).astype(jnp.int32), axis=1,
                      keepdims=True)
        return jnp.where(cnt >= K, cand_u, tu)

    tu = lax.fori_loop(0, 32, bit_body, jnp.zeros((B, 1), jnp.int32))
    ts = tu ^ min32
    gt = key > ts
    eq = key == ts
    cnt_gt = jnp.sum(gt.astype(jnp.int32), axis=1, keepdims=True)
    need = (K - cnt_gt).astype(jnp.float32)
    ii = lax.broadcasted_iota(jnp.int32, (S, S), 0)
    jj = lax.broadcasted_iota(jnp.int32, (S, S), 1)
    tri = (ii <= jj).astype(jnp.float32)
    cum = lax.dot_general(eq.astype(jnp.float32), tri,
                          (((1,), (0,)), ((), ())),
                          preferred_element_type=jnp.float32)
    mask = jnp.logical_or(gt, jnp.logical_and(eq, cum <= need))
    mask_ref[...] = mask.astype(jnp.float32)


def _blend_body(x_hbm, bg_hbm, seg_hbm, mask_hbm, out_hbm,
                xbuf, bbuf, sbuf, rbuf, abuf, mask_v,
                semi0, semi1, semo0, semo1):
    wid = lax.axis_index("s") * NC + lax.axis_index("c")
    row0 = wid * RPW
    in_sems = (semi0, semi1)
    out_sems = (semo0, semo1)

    pltpu.sync_copy(mask_hbm.at[pl.ds(row0, RPW)], mask_v)

    def issue_in(k):
        r, j = divmod(k, CPR)
        slot = k % 2
        sl = pl.ds(j * C, C)
        row = row0 + r
        return (
            pltpu.async_copy(x_hbm.at[row, sl], xbuf.at[slot],
                             in_sems[slot]),
            pltpu.async_copy(bg_hbm.at[row, sl], bbuf.at[slot],
                             in_sems[slot]),
            pltpu.async_copy(seg_hbm.at[row, sl], sbuf.at[slot],
                             in_sems[slot]),
        )

    def issue_out(k):
        r, j = divmod(k, CPR)
        slot = k % 2
        sl = pl.ds(j * C, C)
        row = row0 + r
        return (
            pltpu.async_copy(rbuf.at[slot], out_hbm.at[row, sl],
                             out_sems[slot]),
            pltpu.async_copy(abuf.at[slot], out_hbm.at[B + row, sl],
                             out_sems[slot]),
        )

    in_descs = {0: issue_in(0)}
    out_descs = {}
    for k in range(CPW):
        if k + 1 < CPW:
            in_descs[k + 1] = issue_in(k + 1)
        for d in in_descs.pop(k):
            d.wait()
        if k - 2 in out_descs:
            for d in out_descs.pop(k - 2):
                d.wait()
        slot = k % 2
        r = k // CPR
        rv = jnp.full((L,), r, jnp.int32)
        xb, bb, sb = xbuf.at[slot], bbuf.at[slot], sbuf.at[slot]
        rb, ab_ = rbuf.at[slot], abuf.at[slot]

        @plsc.parallel_loop(0, C, L, unroll=8)
        def _vb(i, xb=xb, bb=bb, sb=sb, rb=rb, ab_=ab_, rv=rv):
            sl = pl.ds(i, L)
            xv = xb[sl]
            bv = bb[sl]
            m = plsc.load_gather(mask_v, [rv, sb[sl]])
            d = (xv - bv) * m
            rb[sl] = bv + d
            ab_[sl] = xv - d
        out_descs[k] = issue_out(k)

    for k in sorted(out_descs):
        for d in out_descs.pop(k):
            d.wait()


def kernel(x, grads, background, seg):
    mesh = plsc.VectorSubcoreMesh(core_axis_name="c", subcore_axis_name="s",
                                  num_cores=NC, num_subcores=NS)
    sc_params = pltpu.CompilerParams(use_tc_tiling_on_sc=False,
                                     needs_layout_passes=False)
    segsum = pl.kernel(
        _segsum_body,
        out_type=jax.ShapeDtypeStruct((B, S), jnp.float32),
        mesh=mesh,
        compiler_params=sc_params,
        scratch_types=[
            pltpu.VMEM((2, C), jnp.float32),
            pltpu.VMEM((2, C), jnp.int32),
            pltpu.VMEM((S,), jnp.float32),
            pltpu.SemaphoreType.DMA,
            pltpu.SemaphoreType.DMA,
        ],
    )
    scores = segsum(grads, seg)

    seg_mask = pl.pallas_call(
        _topk_mask_body,
        out_shape=jax.ShapeDtypeStruct((B, S), jnp.float32),
    )(scores)

    blend = pl.kernel(
        _blend_body,
        out_type=jax.ShapeDtypeStruct((2 * B, N), jnp.float32),
        mesh=mesh,
        compiler_params=sc_params,
        scratch_types=[
            pltpu.VMEM((2, C), jnp.float32),
            pltpu.VMEM((2, C), jnp.float32),
            pltpu.VMEM((2, C), jnp.int32),
            pltpu.VMEM((2, C), jnp.float32),
            pltpu.VMEM((2, C), jnp.float32),
            pltpu.VMEM((RPW, S), jnp.float32),
            pltpu.SemaphoreType.DMA,
            pltpu.SemaphoreType.DMA,
            pltpu.SemaphoreType.DMA,
            pltpu.SemaphoreType.DMA,
        ],
    )
    out = blend(x, background, seg, seg_mask)
    return out.reshape(2, B, N)


# speedup vs baseline: 287.8311x; 1.0075x over previous
import jax
import jax.numpy as jnp
from jax import lax
from jax.experimental import pallas as pl
from jax.experimental.pallas import tpu as pltpu
from jax.experimental.pallas import tpu_sc as plsc

B = 128
N = 32768
S = 1024
K = 50
NC, NS = 2, 16
NW = NC * NS
RPW = B // NW
L = 16
CPR = 4
C = N // CPR
CPW = RPW * CPR
MIN32 = -(2 ** 31)


def _fused_body(grads_hbm, seg_hbm, x_hbm, bg_hbm, out_hbm,
                abuf, bbuf, cbuf, dbuf, ebuf,
                scores_v, keys_v, mask_v,
                sem0, sem1, semo0, semo1):
    wid = lax.axis_index("s") * NC + lax.axis_index("c")
    row0 = wid * RPW
    in_sems = (sem0, sem1)
    out_sems = (semo0, semo1)

    @plsc.parallel_loop(0, RPW * S, L, unroll=8)
    def _zero(i):
        scores_v[pl.ds(i, L)] = jnp.zeros((L,), jnp.float32)

    def issue_a(k):
        r, j = divmod(k, CPR)
        slot = k % 2
        sl = pl.ds(j * C, C)
        return (
            pltpu.async_copy(grads_hbm.at[row0 + r, sl], abuf.at[slot],
                             in_sems[slot]),
            pltpu.async_copy(seg_hbm.at[row0 + r, sl], bbuf.at[slot],
                             in_sems[slot]),
        )

    descs = {0: issue_a(0)}
    for k in range(CPW):
        if k + 1 < CPW:
            descs[k + 1] = issue_a(k + 1)
        for d in descs.pop(k):
            d.wait()
        slot = k % 2
        r = k // CPR
        off = jnp.full((L,), r * S, jnp.int32)
        gb, sb = abuf.at[slot], bbuf.at[slot]

        @plsc.parallel_loop(0, C, L, unroll=8)
        def _acc(i, gb=gb, sb=sb, off=off):
            sl = pl.ds(i, L)
            plsc.addupdate_scatter(scores_v, [sb[sl] + off], gb[sl])

    def issue_c(k):
        r, j = divmod(k, CPR)
        slot = k % 2
        sl = pl.ds(j * C, C)
        row = row0 + r
        return (
            pltpu.async_copy(x_hbm.at[row, sl], abuf.at[slot],
                             in_sems[slot]),
            pltpu.async_copy(bg_hbm.at[row, sl], cbuf.at[slot],
                             in_sems[slot]),
            pltpu.async_copy(seg_hbm.at[row, sl], bbuf.at[slot],
                             in_sems[slot]),
        )

    c_descs = {0: issue_c(0)}

    kv = jnp.int32(K)
    min32 = jnp.full((L,), MIN32, jnp.int32)
    m7f = jnp.full((L,), 0x7FFFFFFF, jnp.int32)
    one = jnp.full((L,), 1, jnp.int32)
    for r in range(RPW):
        sc_r = scores_v.at[pl.ds(r * S, S)]
        mk_r = mask_v.at[pl.ds(r * S, S)]

        @plsc.parallel_loop(0, S, L, unroll=8)
        def _mkkeys(i, sc_r=sc_r):
            v = plsc.bitcast(sc_r[pl.ds(i, L)], jnp.int32)
            keys_v[pl.ds(i, L)] = v ^ (jnp.right_shift(v, 31) & m7f)

        def bit_body(it, tu):
            bitv = jnp.left_shift(one, 31 - it)
            cand_u = tu | bitv
            cand_s = cand_u ^ min32

            @plsc.parallel_loop(0, S, L, unroll=8,
                                carry=jnp.zeros((L,), jnp.int32))
            def cnt(i, c, cand_s=cand_s):
                ge = keys_v[pl.ds(i, L)] >= cand_s
                return c + plsc.all_reduce_population_count(ge)
            return jnp.where(cnt >= kv, cand_u, tu)

        tu = lax.fori_loop(0, 32, bit_body, jnp.zeros((L,), jnp.int32))
        ts = tu ^ min32
        ts1 = ts + 1

        @plsc.parallel_loop(0, S, L, unroll=8,
                            carry=jnp.zeros((L,), jnp.int32))
        def cnt_gt(i, c, ts1=ts1):
            ge = keys_v[pl.ds(i, L)] >= ts1
            return c + plsc.all_reduce_population_count(ge)

        need = jnp.int32(K) - cnt_gt

        @plsc.parallel_loop(0, S, L, unroll=8,
                            carry=jnp.zeros((L,), jnp.int32))
        def _mkmask(i, c, ts=ts, need=need, mk_r=mk_r):
            k16 = keys_v[pl.ds(i, L)]
            eq = k16 == ts
            gt = k16 > ts
            eqi = jnp.where(eq, 1, 0).astype(jnp.int32)
            cum = plsc.cumsum(eqi) + c
            sel = jnp.logical_or(gt, jnp.logical_and(eq, cum <= need))
            mk_r[pl.ds(i, L)] = jnp.where(sel, 1.0, 0.0).astype(jnp.float32)
            return c + plsc.all_reduce_population_count(eq)

    def issue_out(k):
        r, j = divmod(k, CPR)
        slot = k % 2
        sl = pl.ds(j * C, C)
        row = row0 + r
        return (
            pltpu.async_copy(dbuf.at[slot], out_hbm.at[row, sl],
                             out_sems[slot]),
            pltpu.async_copy(ebuf.at[slot], out_hbm.at[B + row, sl],
                             out_sems[slot]),
        )

    out_descs = {}
    for k in range(CPW):
        if k + 1 < CPW:
            c_descs[k + 1] = issue_c(k + 1)
        for d in c_descs.pop(k):
            d.wait()
        if k - 2 in out_descs:
            for d in out_descs.pop(k - 2):
                d.wait()
        slot = k % 2
        r = k // CPR
        off = jnp.full((L,), r * S, jnp.int32)
        xb, sb, bb = abuf.at[slot], bbuf.at[slot], cbuf.at[slot]
        rb, ab_ = dbuf.at[slot], ebuf.at[slot]

        @plsc.parallel_loop(0, C, L, unroll=8)
        def _vb(i, xb=xb, bb=bb, sb=sb, rb=rb, ab_=ab_, off=off):
            sl = pl.ds(i, L)
            xv = xb[sl]
            bv = bb[sl]
            m = plsc.load_gather(mask_v, [sb[sl] + off])
            d = (xv - bv) * m
            rb[sl] = bv + d
            ab_[sl] = xv - d
        out_descs[k] = issue_out(k)

    for k in sorted(out_descs):
        for d in out_descs.pop(k):
            d.wait()


def kernel(x, grads, background, seg):
    mesh = plsc.VectorSubcoreMesh(core_axis_name="c", subcore_axis_name="s",
                                  num_cores=NC, num_subcores=NS)
    sc_params = pltpu.CompilerParams(use_tc_tiling_on_sc=False,
                                     needs_layout_passes=False)
    fused = pl.kernel(
        _fused_body,
        out_type=jax.ShapeDtypeStruct((2 * B, N), jnp.float32),
        mesh=mesh,
        compiler_params=sc_params,
        scratch_types=[
            pltpu.VMEM((2, C), jnp.float32),
            pltpu.VMEM((2, C), jnp.int32),
            pltpu.VMEM((2, C), jnp.float32),
            pltpu.VMEM((2, C), jnp.float32),
            pltpu.VMEM((2, C), jnp.float32),
            pltpu.VMEM((RPW * S,), jnp.float32),
            pltpu.VMEM((S,), jnp.int32),
            pltpu.VMEM((RPW * S,), jnp.float32),
            pltpu.SemaphoreType.DMA,
            pltpu.SemaphoreType.DMA,
            pltpu.SemaphoreType.DMA,
            pltpu.SemaphoreType.DMA,
        ],
    )
    out = fused(grads, seg, x, background)
    return out.reshape(2, B, N)


# speedup vs baseline: 620.9128x; 2.1572x over previous
import jax
import jax.numpy as jnp
from jax import lax
from jax.experimental import pallas as pl
from jax.experimental.pallas import tpu as pltpu
from jax.experimental.pallas import tpu_sc as plsc

B = 128
N = 32768
S = 1024
K = 50
NC, NS = 2, 16
L = 16
RB = 8
HW = N // 2
CW = 1024
CPH = HW // CW
RPT = RB // 2
MIN32 = -(2 ** 31)


def _fused_body(grads_hbm, seg_hbm, x_hbm, bg_hbm, out_hbm,
                abuf, bbuf, cbuf, dbuf, ebuf,
                scores_v, tmp_v, keys_v, mask_v, spmem,
                sem0, sem1, semo0, semo1):
    s_idx = lax.axis_index("s")
    c_idx = lax.axis_index("c")
    rb = c_idx * (NS // 2) + s_idx // 2
    half = s_idx % 2
    col0 = half * HW
    rows = pl.ds(rb * RB, RB)
    in_sems = (sem0, sem1)
    out_sems = (semo0, semo1)

    @plsc.parallel_loop(0, RB * S, L, unroll=8)
    def _zero(i):
        scores_v[pl.ds(i, L)] = jnp.zeros((L,), jnp.float32)

    def issue_a(k):
        slot = k % 2
        cols = pl.ds(col0 + k * CW, CW)
        return (
            pltpu.async_copy(grads_hbm.at[rows, cols], abuf.at[slot],
                             in_sems[slot]),
            pltpu.async_copy(seg_hbm.at[rows, cols], bbuf.at[slot],
                             in_sems[slot]),
        )

    descs = {0: issue_a(0)}
    for k in range(CPH):
        if k + 1 < CPH:
            descs[k + 1] = issue_a(k + 1)
        for d in descs.pop(k):
            d.wait()
        slot = k % 2
        gb, sb = abuf.at[slot], bbuf.at[slot]

        def arow(r, c2, gb=gb, sb=sb):
            off = jnp.broadcast_to(r * S, (L,)).astype(jnp.int32)

            @plsc.parallel_loop(0, CW, L, unroll=8)
            def _acc(i, off=off, r=r, gb=gb, sb=sb):
                sl = pl.ds(i, L)
                plsc.addupdate_scatter(scores_v, [sb[r, sl] + off],
                                       gb[r, sl])
            return c2
        lax.fori_loop(0, RB, arow, 0)

    pltpu.sync_copy(scores_v, spmem.at[s_idx])
    plsc.subcore_barrier()
    pltpu.sync_copy(spmem.at[s_idx ^ 1], tmp_v)
    plsc.subcore_barrier()

    @plsc.parallel_loop(0, RB * S, L, unroll=8)
    def _merge(i):
        sl = pl.ds(i, L)
        scores_v[sl] = scores_v[sl] + tmp_v[sl]

    def issue_d(k):
        slot = k % 2
        cols = pl.ds(col0 + k * CW, CW)
        return (
            pltpu.async_copy(x_hbm.at[rows, cols], abuf.at[slot],
                             in_sems[slot]),
            pltpu.async_copy(bg_hbm.at[rows, cols], cbuf.at[slot],
                             in_sems[slot]),
            pltpu.async_copy(seg_hbm.at[rows, cols], bbuf.at[slot],
                             in_sems[slot]),
        )

    d_descs = {0: issue_d(0)}

    kv = jnp.int32(K)
    min32 = jnp.full((L,), MIN32, jnp.int32)
    m7f = jnp.full((L,), 0x7FFFFFFF, jnp.int32)
    one = jnp.full((L,), 1, jnp.int32)
    my_r0 = half * RPT
    for rr in range(RPT):
        r = my_r0 + rr
        sc_r = scores_v.at[pl.ds(r * S, S)]
        mk_r = mask_v.at[pl.ds(r * S, S)]

        @plsc.parallel_loop(0, S, L, unroll=8)
        def _mkkeys(i, sc_r=sc_r):
            v = plsc.bitcast(sc_r[pl.ds(i, L)], jnp.int32)
            keys_v[pl.ds(i, L)] = v ^ (jnp.right_shift(v, 31) & m7f)

        def bit_body(it, tu):
            bitv = jnp.left_shift(one, 31 - it)
            cand_u = tu | bitv
            cand_s = cand_u ^ min32

            @plsc.parallel_loop(0, S, L, unroll=8,
                                carry=jnp.zeros((L,), jnp.int32))
            def cnt(i, c, cand_s=cand_s):
                ge = keys_v[pl.ds(i, L)] >= cand_s
                return c + plsc.all_reduce_population_count(ge)
            return jnp.where(cnt >= kv, cand_u, tu)

        tu = lax.fori_loop(0, 32, bit_body, jnp.zeros((L,), jnp.int32))
        ts = tu ^ min32
        ts1 = ts + 1

        @plsc.parallel_loop(0, S, L, unroll=8,
                            carry=jnp.zeros((L,), jnp.int32))
        def cnt_gt(i, c, ts1=ts1):
            ge = keys_v[pl.ds(i, L)] >= ts1
            return c + plsc.all_reduce_population_count(ge)

        need = jnp.int32(K) - cnt_gt

        @plsc.parallel_loop(0, S, L, unroll=8,
                            carry=jnp.zeros((L,), jnp.int32))
        def _mkmask(i, c, ts=ts, need=need, mk_r=mk_r):
            k16 = keys_v[pl.ds(i, L)]
            eq = k16 == ts
            gt = k16 > ts
            eqi = jnp.where(eq, 1, 0).astype(jnp.int32)
            cum = plsc.cumsum(eqi) + c
            sel = jnp.logical_or(gt, jnp.logical_and(eq, cum <= need))
            mk_r[pl.ds(i, L)] = jnp.where(sel, 1.0, 0.0).astype(jnp.float32)
            return c + plsc.all_reduce_population_count(eq)

    pltpu.sync_copy(mask_v.at[pl.ds(my_r0 * S, RPT * S)],
                    spmem.at[s_idx, pl.ds(my_r0 * S, RPT * S)])
    plsc.subcore_barrier()
    ot_r0 = (1 - half) * RPT
    pltpu.sync_copy(spmem.at[s_idx ^ 1, pl.ds(ot_r0 * S, RPT * S)],
                    mask_v.at[pl.ds(ot_r0 * S, RPT * S)])

    def issue_out(k):
        slot = k % 2
        cols = pl.ds(col0 + k * CW, CW)
        return (
            pltpu.async_copy(dbuf.at[slot], out_hbm.at[rows, cols],
                             out_sems[slot]),
            pltpu.async_copy(ebuf.at[slot],
                             out_hbm.at[pl.ds(B + rb * RB, RB), cols],
                             out_sems[slot]),
        )

    out_descs = {}
    for k in range(CPH):
        if k + 1 < CPH:
            d_descs[k + 1] = issue_d(k + 1)
        for d in d_descs.pop(k):
            d.wait()
        if k - 2 in out_descs:
            for d in out_descs.pop(k - 2):
                d.wait()
        slot = k % 2
        xb, sb, bb = abuf.at[slot], bbuf.at[slot], cbuf.at[slot]
        rbf, af = dbuf.at[slot], ebuf.at[slot]

        def drow(r, c2, xb=xb, bb=bb, sb=sb, rbf=rbf, af=af):
            off = jnp.broadcast_to(r * S, (L,)).astype(jnp.int32)

            @plsc.parallel_loop(0, CW, L, unroll=8)
            def _vb(i, off=off, r=r, xb=xb, bb=bb, sb=sb, rbf=rbf, af=af):
                sl = pl.ds(i, L)
                xv = xb[r, sl]
                bv = bb[r, sl]
                m = plsc.load_gather(mask_v, [sb[r, sl] + off])
                d = (xv - bv) * m
                rbf[r, sl] = bv + d
                af[r, sl] = xv - d
            return c2
        lax.fori_loop(0, RB, drow, 0)
        out_descs[k] = issue_out(k)

    for k in sorted(out_descs):
        for d in out_descs.pop(k):
            d.wait()


def kernel(x, grads, background, seg):
    mesh = plsc.VectorSubcoreMesh(core_axis_name="c", subcore_axis_name="s",
                                  num_cores=NC, num_subcores=NS)
    sc_params = pltpu.CompilerParams(use_tc_tiling_on_sc=True,
                                     needs_layout_passes=False)
    fused = pl.kernel(
        _fused_body,
        out_type=jax.ShapeDtypeStruct((2 * B, N), jnp.float32),
        mesh=mesh,
        compiler_params=sc_params,
        scratch_types=[
            pltpu.VMEM((2, RB, CW), jnp.float32),
            pltpu.VMEM((2, RB, CW), jnp.int32),
            pltpu.VMEM((2, RB, CW), jnp.float32),
            pltpu.VMEM((2, RB, CW), jnp.float32),
            pltpu.VMEM((2, RB, CW), jnp.float32),
            pltpu.VMEM((RB * S,), jnp.float32),
            pltpu.VMEM((RB * S,), jnp.float32),
            pltpu.VMEM((S,), jnp.int32),
            pltpu.VMEM((RB * S,), jnp.float32),
            pltpu.VMEM_SHARED((NS, RB * S), jnp.float32),
            pltpu.SemaphoreType.DMA,
            pltpu.SemaphoreType.DMA,
            pltpu.SemaphoreType.DMA,
            pltpu.SemaphoreType.DMA,
        ],
    )
    out = fused(grads, seg, x, background)
    return out.reshape(2, B, N)
